# speedup vs baseline: 5.7052x; 5.7052x over previous
import functools

import jax
import jax.numpy as jnp
from jax import lax
from jax.experimental import pallas as pl
from jax.experimental.pallas import tpu as pltpu
from jax.experimental.pallas import tpu_sc as plsc

DIM = 128
LANES = 16


def _make_kernel(n_rows: int, chunk: int):
    info = plsc.get_sparse_core_info()
    nc, ns = info.num_cores, info.num_subcores
    nw = nc * ns
    rows_per_w = n_rows // nw
    n_chunks = rows_per_w // chunk
    mesh = plsc.VectorSubcoreMesh(core_axis_name="c", subcore_axis_name="s")

    @functools.partial(
        pl.kernel,
        mesh=mesh,
        out_type=jax.ShapeDtypeStruct((n_rows, DIM), jnp.float32),
        scratch_types=[
            pltpu.VMEM((chunk,), jnp.int32),
            pltpu.VMEM((chunk,), jnp.int32),
            pltpu.VMEM((chunk, DIM), jnp.float32),
            pltpu.VMEM((chunk, DIM), jnp.float32),
            pltpu.SemaphoreType.DMA,
            pltpu.SemaphoreType.DMA,
        ],
    )
    def k(tok_hbm, pos_hbm, emb_hbm, ptab_hbm, out_hbm,
          tok_v, pos_v, erows_v, prows_v, esem, psem):
        wid = lax.axis_index("s") * nc + lax.axis_index("c")
        wbase = wid * rows_per_w

        def chunk_body(g, _):
            base = wbase + g * chunk
            pltpu.sync_copy(tok_hbm.at[pl.ds(base, chunk)], tok_v)
            pltpu.sync_copy(pos_hbm.at[pl.ds(base, chunk)], pos_v)
            ecp = pltpu.async_copy(emb_hbm.at[tok_v], erows_v, esem)
            pcp = pltpu.async_copy(ptab_hbm.at[pos_v], prows_v, psem)
            ecp.wait()
            pcp.wait()

            def row_body(r, _):
                for j in range(DIM // LANES):
                    sl = (r, pl.ds(j * LANES, LANES))
                    erows_v[sl] = erows_v[sl] + prows_v[sl]
                return 0

            lax.fori_loop(0, chunk, row_body, 0)
            pltpu.sync_copy(erows_v, out_hbm.at[pl.ds(base, chunk)])
            return 0

        lax.fori_loop(0, n_chunks, chunk_body, 0)

    return k


def kernel(input_tensor, incremental_mask, emb_table, pos_table):
    b, l = input_tensor.shape
    n = b * l
    tok = input_tensor.reshape(n)
    pos = incremental_mask.reshape(n)
    out = _make_kernel(n, 128)(tok, pos, emb_table, pos_table)
    return out.reshape(b, l, DIM)


# speedup vs baseline: 7.6320x; 1.3377x over previous
import functools

import jax
import jax.numpy as jnp
from jax import lax
from jax.experimental import pallas as pl
from jax.experimental.pallas import tpu as pltpu
from jax.experimental.pallas import tpu_sc as plsc

DIM = 128
LANES = 16


def _make_kernel(n_rows: int, chunk: int):
    info = plsc.get_sparse_core_info()
    nc, ns = info.num_cores, info.num_subcores
    nw = nc * ns
    rows_per_w = n_rows // nw
    n_chunks = rows_per_w // chunk
    n_pairs = n_chunks // 2
    mesh = plsc.VectorSubcoreMesh(core_axis_name="c", subcore_axis_name="s")

    @functools.partial(
        pl.kernel,
        mesh=mesh,
        out_type=jax.ShapeDtypeStruct((n_rows, DIM), jnp.float32),
        scratch_types=[
            pltpu.VMEM((2, chunk), jnp.int32),
            pltpu.VMEM((2, chunk), jnp.int32),
            pltpu.VMEM((2, chunk, DIM), jnp.float32),
            pltpu.VMEM((2, chunk, DIM), jnp.float32),
            pltpu.VMEM((2, chunk, DIM), jnp.float32),
            pltpu.SemaphoreType.DMA,
            pltpu.SemaphoreType.DMA,
            pltpu.SemaphoreType.DMA,
            pltpu.SemaphoreType.DMA,
            pltpu.SemaphoreType.DMA,
            pltpu.SemaphoreType.DMA,
            pltpu.SemaphoreType.DMA,
            pltpu.SemaphoreType.DMA,
        ],
    )
    def k(tok_hbm, pos_hbm, emb_hbm, ptab_hbm, out_hbm,
          tok_v, pos_v, erows_v, prows_v, obuf_v,
          isem0, isem1, esem0, esem1, psem0, psem1, wsem0, wsem1):
        isem = (isem0, isem1)
        esem = (esem0, esem1)
        psem = (psem0, psem1)
        wsem = (wsem0, wsem1)
        wid = lax.axis_index("s") * nc + lax.axis_index("c")
        wbase = wid * rows_per_w

        def fire_idx(g, b):
            base = wbase + g * chunk
            pltpu.async_copy(tok_hbm.at[pl.ds(base, chunk)], tok_v.at[b], isem[b])
            pltpu.async_copy(pos_hbm.at[pl.ds(base, chunk)], pos_v.at[b], isem[b])

        def wait_idx(b):
            pltpu.make_async_copy(tok_hbm.at[pl.ds(0, chunk)], tok_v.at[b], isem[b]).wait()
            pltpu.make_async_copy(pos_hbm.at[pl.ds(0, chunk)], pos_v.at[b], isem[b]).wait()

        def fire_gather(b):
            pltpu.async_copy(emb_hbm.at[tok_v.at[b]], erows_v.at[b], esem[b])
            pltpu.async_copy(ptab_hbm.at[pos_v.at[b]], prows_v.at[b], psem[b])

        def wait_gather(b):
            pltpu.make_async_copy(emb_hbm.at[tok_v.at[b]], erows_v.at[b], esem[b]).wait()
            pltpu.make_async_copy(ptab_hbm.at[pos_v.at[b]], prows_v.at[b], psem[b]).wait()

        def fire_wb(g, b):
            base = wbase + g * chunk
            pltpu.async_copy(obuf_v.at[b], out_hbm.at[pl.ds(base, chunk)], wsem[b])

        def wait_wb(b):
            pltpu.make_async_copy(obuf_v.at[b], out_hbm.at[pl.ds(0, chunk)], wsem[b]).wait()

        def add(b):
            @plsc.parallel_loop(0, chunk, unroll=2)
            def _row(r):
                for j in range(DIM // LANES):
                    sl = pl.ds(j * LANES, LANES)
                    obuf_v[b, r, sl] = erows_v[b, r, sl] + prows_v[b, r, sl]

        fire_idx(0, 0)
        fire_idx(1, 1)
        wait_idx(0)
        fire_gather(0)
        wait_idx(1)
        fire_gather(1)

        def pair_body(h, _):
            for b in range(2):
                g = 2 * h + b

                wait_gather(b)

                @pl.when(h < n_pairs - 1)
                def _prefetch_idx():
                    fire_idx(g + 2, b)

                @pl.when(h > 0)
                def _drain_wb():
                    wait_wb(b)

                add(b)
                fire_wb(g, b)

                @pl.when(h < n_pairs - 1)
                def _next_gather():
                    wait_idx(b)
                    fire_gather(b)

            return 0

        lax.fori_loop(0, n_pairs, pair_body, 0)
        wait_wb(0)
        wait_wb(1)

    return k


def kernel(input_tensor, incremental_mask, emb_table, pos_table):
    b, l = input_tensor.shape
    n = b * l
    tok = input_tensor.reshape(n)
    pos = incremental_mask.reshape(n)
    out = _make_kernel(n, 128)(tok, pos, emb_table, pos_table)
    return out.reshape(b, l, DIM)
